# baseline (speedup 1.0000x reference)
import functools

import jax
import jax.numpy as jnp
from jax import lax
from jax.experimental import pallas as pl
from jax.experimental.pallas import tpu as pltpu
from jax.experimental.pallas import tpu_sc as plsc

_CHR = 4


@functools.cache
def _build(b, h, d):
    info = plsc.get_sparse_core_info()
    nw = info.num_cores * info.num_subcores
    rows_w = b // nw
    n_ch = rows_w // _CHR
    n_pair = n_ch // 2
    g0 = 128
    g1 = h - g0

    mesh = plsc.VectorSubcoreMesh(core_axis_name="c", subcore_axis_name="s")

    @functools.partial(
        pl.kernel,
        out_type=jax.ShapeDtypeStruct((b, h, d), jnp.float32),
        mesh=mesh,
        compiler_params=pltpu.CompilerParams(use_tc_tiling_on_sc=False),
        scratch_types=[
            pltpu.VMEM((rows_w, h), jnp.int32),
            pltpu.VMEM((_CHR, h, d), jnp.float32),
            pltpu.VMEM((_CHR, h, d), jnp.float32),
            pltpu.SemaphoreType.DMA,
            pltpu.SemaphoreType.DMA,
            pltpu.SemaphoreType.DMA,
        ],
    )
    def emb(x_hbm, w_hbm, out_hbm, idx_v, buf0, buf1, gsem0, gsem1, osem):
        wid = lax.axis_index("s") * info.num_cores + lax.axis_index("c")
        b0 = wid * rows_w
        pltpu.sync_copy(x_hbm.at[pl.ds(b0, rows_w)], idx_v)

        def fire(c, buf, sem):
            for i in range(_CHR):
                r = c * _CHR + i
                pltpu.async_copy(
                    w_hbm.at[idx_v.at[r, pl.ds(0, g0)]],
                    buf.at[i, pl.ds(0, g0)],
                    sem,
                )
                pltpu.async_copy(
                    w_hbm.at[idx_v.at[r, pl.ds(g0, g1)]],
                    buf.at[i, pl.ds(g0, g1)],
                    sem,
                )

        def drain(buf, sem):
            pltpu.make_async_copy(out_hbm.at[pl.ds(0, _CHR)], buf, sem).wait()

        def outcopy(c, buf):
            pltpu.async_copy(buf, out_hbm.at[pl.ds(b0 + c * _CHR, _CHR)], osem)

        def owait():
            pltpu.make_async_copy(buf0, out_hbm.at[pl.ds(b0, _CHR)], osem).wait()

        fire(0, buf0, gsem0)
        fire(1, buf1, gsem1)
        drain(buf0, gsem0)
        outcopy(0, buf0)

        def body(u, carry):
            a = 2 * u
            owait()
            fire(a, buf0, gsem0)
            drain(buf1, gsem1)
            outcopy(a - 1, buf1)
            owait()
            fire(a + 1, buf1, gsem1)
            drain(buf0, gsem0)
            outcopy(a, buf0)
            return carry

        lax.fori_loop(1, n_pair, body, 0)

        owait()
        drain(buf1, gsem1)
        outcopy(n_ch - 1, buf1)
        owait()

    return emb


def kernel(x, weight):
    b, h = x.shape
    _, d = weight.shape
    return _build(b, h, d)(x.astype(jnp.int32), weight)
